# baseline (speedup 1.0000x reference)
import functools

import jax
import jax.numpy as jnp
from jax import lax
from jax.experimental import pallas as pl
from jax.experimental.pallas import tpu as pltpu
from jax.experimental.pallas import tpu_sc as plsc

N = 20000
C = 80
CG = 40
K = 1000
KPAD = 1024
NW = 16
KT = KPAD // NW
L = 16
BIG = 2 ** 30
NP = 20096
NT = 6


def _dense_body(scores_ref, boxes_ref, idx_ref,
                tab_ref, pidx_ref,
                am_ref, aa_ref):
    g = pl.program_id(0)
    blk = scores_ref[0]
    m_g = jnp.max(blk, axis=0)
    rows = lax.broadcasted_iota(jnp.int32, (CG, N), 0) + CG * g
    cls_g = jnp.min(jnp.where(blk == m_g[None, :], rows, BIG), axis=0)

    @pl.when(g == 0)
    def _():
        am_ref[...] = m_g
        aa_ref[...] = cls_g
        for c in range(4):
            tab_ref[pl.ds((2 + c) * NP, N)] = boxes_ref[0, c, :]
        col = idx_ref[0, :, 2]
        pidx_ref[...] = jnp.concatenate(
            [col, jnp.zeros((KPAD - K,), jnp.int32)])

    @pl.when(g > 0)
    def _():
        m0 = am_ref[...]
        better = m_g > m0
        aa_ref[...] = jnp.where(better, cls_g, aa_ref[...])
        am_ref[...] = jnp.where(better, m_g, m0)

    @pl.when(g == C // CG - 1)
    def _():
        tab_ref[pl.ds(0, N)] = am_ref[...]
        tab_ref[pl.ds(NP, N)] = lax.bitcast_convert_type(aa_ref[...],
                                                         jnp.float32)


def _dense_tc(idxTensor, boxes, scores):
    return pl.pallas_call(
        _dense_body,
        grid=(C // CG,),
        in_specs=[
            pl.BlockSpec((1, CG, N), lambda g: (0, g, 0)),
            pl.BlockSpec((1, 4, N), lambda g: (0, 0, 0)),
            pl.BlockSpec((1, K, 3), lambda g: (0, 0, 0)),
        ],
        out_specs=[pl.BlockSpec((NT * NP,), lambda g: (0,)),
                   pl.BlockSpec((KPAD,), lambda g: (0,))],
        out_shape=[
            jax.ShapeDtypeStruct((NT * NP,), jnp.float32),
            jax.ShapeDtypeStruct((KPAD,), jnp.int32),
        ],
        scratch_shapes=[
            pltpu.VMEM((N,), jnp.float32),
            pltpu.VMEM((N,), jnp.int32),
        ],
    )(scores, boxes, idxTensor[None])


_mesh = plsc.VectorSubcoreMesh(core_axis_name="c", subcore_axis_name="s",
                               num_cores=1)


@functools.partial(
    pl.kernel,
    mesh=_mesh,
    out_type=[
        jax.ShapeDtypeStruct((4, KPAD), jnp.float32),
        jax.ShapeDtypeStruct((KPAD,), jnp.float32),
        jax.ShapeDtypeStruct((KPAD,), jnp.float32),
    ],
    scratch_types=[
        pltpu.VMEM((KT,), jnp.int32),
        pltpu.VMEM((3, 128), jnp.int32),
        pltpu.VMEM((NT * KT,), jnp.float32),
        pltpu.VMEM((4, KT), jnp.float32),
        pltpu.SemaphoreType.DMA,
        pltpu.SemaphoreType.DMA,
    ],
)
def _gather_sc(idx_hbm, tab_hbm,
               bbox_hbm, score_hbm, cls_hbm,
               idx_v, fidx_v, gv, bb_v, sem, osem):
    base = lax.axis_index("s") * KT
    pltpu.sync_copy(idx_hbm.at[pl.ds(base, KT)], idx_v)
    iv = [idx_v[pl.ds(h * L, L)] for h in range(KT // L)]
    for r in range(3):
        for q in range(128 // L):
            p = r * 128 + q * L
            fidx_v[r, pl.ds(q * L, L)] = iv[(p % KT) // L] + (p // KT) * NP
    copies = [
        pltpu.async_copy(tab_hbm.at[fidx_v.at[r]],
                         gv.at[pl.ds(r * 128, 128)], sem)
        for r in range(3)
    ]
    copies[2].wait()
    copies[1].wait()
    for h in range(KT // L):
        s = pl.ds(h * L, L)
        cx = gv[pl.ds(2 * KT + h * L, L)]
        cy = gv[pl.ds(3 * KT + h * L, L)]
        w = gv[pl.ds(4 * KT + h * L, L)]
        hh = gv[pl.ds(5 * KT + h * L, L)]
        bb_v[0, s] = (cx - 0.5 * w) / 640.0
        bb_v[1, s] = (cy - 0.5 * hh) / 640.0
        bb_v[2, s] = (cx + 0.5 * w) / 640.0
        bb_v[3, s] = (cy + 0.5 * hh) / 640.0
    out = [
        pltpu.async_copy(bb_v.at[c], bbox_hbm.at[c, pl.ds(base, KT)], osem)
        for c in range(4)
    ]
    copies[0].wait()
    out.append(pltpu.async_copy(gv.at[pl.ds(0, KT)],
                                score_hbm.at[pl.ds(base, KT)], osem))
    out.append(pltpu.async_copy(gv.at[pl.ds(KT, KT)],
                                cls_hbm.at[pl.ds(base, KT)], osem))
    for cp in out:
        cp.wait()


def kernel(idxTensor, boxes, scores):
    tab, idx = _dense_tc(idxTensor.astype(jnp.int32), boxes, scores)
    bb, sc, clf = _gather_sc(idx, tab)
    bbox = jnp.stack([bb[0, :K], bb[1, :K], bb[2, :K], bb[3, :K]], axis=-1)
    cl = lax.bitcast_convert_type(clf[:K], jnp.int32)
    return bbox[None], sc[:K][None], cl[None]


# speedup vs baseline: 1.0550x; 1.0550x over previous
import functools

import jax
import jax.numpy as jnp
from jax import lax
from jax.experimental import pallas as pl
from jax.experimental.pallas import tpu as pltpu
from jax.experimental.pallas import tpu_sc as plsc

N = 20000
C = 80
CG = 40
K = 1000
KPAD = 1024
NW = 16
KT = KPAD // NW
L = 16
BIG = 2 ** 30


def _dense_body(scores_ref, boxes_ref, idx_ref,
                mx_ref, ag_ref, cx_ref, cy_ref, w_ref, h_ref, pidx_ref,
                am_ref, aa_ref):
    g = pl.program_id(0)
    blk = scores_ref[0]
    m_g = jnp.max(blk, axis=0)
    rows = lax.broadcasted_iota(jnp.int32, (CG, N), 0) + CG * g
    cls_g = jnp.min(jnp.where(blk == m_g[None, :], rows, BIG), axis=0)

    @pl.when(g == 0)
    def _():
        am_ref[...] = m_g
        aa_ref[...] = cls_g
        cx_ref[...] = boxes_ref[0, 0, :]
        cy_ref[...] = boxes_ref[0, 1, :]
        w_ref[...] = boxes_ref[0, 2, :]
        h_ref[...] = boxes_ref[0, 3, :]
        col = idx_ref[0, :, 2]
        pidx_ref[...] = jnp.concatenate(
            [col, jnp.zeros((KPAD - K,), jnp.int32)])

    @pl.when(g > 0)
    def _():
        m0 = am_ref[...]
        better = m_g > m0
        aa_ref[...] = jnp.where(better, cls_g, aa_ref[...])
        am_ref[...] = jnp.where(better, m_g, m0)

    @pl.when(g == C // CG - 1)
    def _():
        mx_ref[...] = am_ref[...]
        ag_ref[...] = aa_ref[...]


def _dense_tc(idxTensor, boxes, scores):
    return pl.pallas_call(
        _dense_body,
        grid=(C // CG,),
        in_specs=[
            pl.BlockSpec((1, CG, N), lambda g: (0, g, 0)),
            pl.BlockSpec((1, 4, N), lambda g: (0, 0, 0)),
            pl.BlockSpec((1, K, 3), lambda g: (0, 0, 0)),
        ],
        out_specs=[pl.BlockSpec((N,), lambda g: (0,))] * 6
        + [pl.BlockSpec((KPAD,), lambda g: (0,))],
        out_shape=[
            jax.ShapeDtypeStruct((N,), jnp.float32),
            jax.ShapeDtypeStruct((N,), jnp.int32),
            jax.ShapeDtypeStruct((N,), jnp.float32),
            jax.ShapeDtypeStruct((N,), jnp.float32),
            jax.ShapeDtypeStruct((N,), jnp.float32),
            jax.ShapeDtypeStruct((N,), jnp.float32),
            jax.ShapeDtypeStruct((KPAD,), jnp.int32),
        ],
        scratch_shapes=[
            pltpu.VMEM((N,), jnp.float32),
            pltpu.VMEM((N,), jnp.int32),
        ],
    )(scores, boxes, idxTensor[None])


_mesh = plsc.VectorSubcoreMesh(core_axis_name="c", subcore_axis_name="s",
                               num_cores=1)


@functools.partial(
    pl.kernel,
    mesh=_mesh,
    out_type=[
        jax.ShapeDtypeStruct((4, KPAD), jnp.float32),
        jax.ShapeDtypeStruct((KPAD,), jnp.float32),
        jax.ShapeDtypeStruct((KPAD,), jnp.int32),
    ],
    scratch_types=[
        pltpu.VMEM((KT,), jnp.int32),
        pltpu.VMEM((KT,), jnp.float32),
        pltpu.VMEM((KT,), jnp.int32),
        pltpu.VMEM((4, KT), jnp.float32),
        pltpu.VMEM((4, KT), jnp.float32),
        pltpu.SemaphoreType.DMA,
        pltpu.SemaphoreType.DMA,
    ],
)
def _gather_sc(idx_hbm, mx_hbm, ag_hbm, cx_hbm, cy_hbm, w_hbm, h_hbm,
               bbox_hbm, score_hbm, cls_hbm,
               idx_v, mx_v, ag_v, bx_v, bb_v, sem, osem):
    base = lax.axis_index("s") * KT
    pltpu.sync_copy(idx_hbm.at[pl.ds(base, KT)], idx_v)
    box_copies = [
        pltpu.async_copy(cx_hbm.at[idx_v], bx_v.at[0], sem),
        pltpu.async_copy(cy_hbm.at[idx_v], bx_v.at[1], sem),
        pltpu.async_copy(w_hbm.at[idx_v], bx_v.at[2], sem),
        pltpu.async_copy(h_hbm.at[idx_v], bx_v.at[3], sem),
    ]
    sc_copies = [
        pltpu.async_copy(mx_hbm.at[idx_v], mx_v, osem),
        pltpu.async_copy(ag_hbm.at[idx_v], ag_v, osem),
    ]
    for cp in box_copies:
        cp.wait()
    for h in range(KT // L):
        s = pl.ds(h * L, L)
        cx = bx_v[0, s]
        cy = bx_v[1, s]
        w = bx_v[2, s]
        hh = bx_v[3, s]
        bb_v[0, s] = (cx - 0.5 * w) / 640.0
        bb_v[1, s] = (cy - 0.5 * hh) / 640.0
        bb_v[2, s] = (cx + 0.5 * w) / 640.0
        bb_v[3, s] = (cy + 0.5 * hh) / 640.0
    out = [
        pltpu.async_copy(bb_v.at[c], bbox_hbm.at[c, pl.ds(base, KT)], sem)
        for c in range(4)
    ]
    for cp in sc_copies:
        cp.wait()
    out.append(pltpu.async_copy(mx_v, score_hbm.at[pl.ds(base, KT)], sem))
    out.append(pltpu.async_copy(ag_v, cls_hbm.at[pl.ds(base, KT)], sem))
    for cp in out:
        cp.wait()


def kernel(idxTensor, boxes, scores):
    mx, ag, cx, cy, w, h, idx = _dense_tc(idxTensor.astype(jnp.int32),
                                          boxes, scores)
    bb, sc, cl = _gather_sc(idx, mx, ag, cx, cy, w, h)
    bbox = jnp.stack([bb[0, :K], bb[1, :K], bb[2, :K], bb[3, :K]], axis=-1)
    return bbox[None], sc[:K][None], cl[:K][None]
